# baseline (speedup 1.0000x reference)
import dataclasses
import functools

import jax
import jax.numpy as jnp
from jax import lax
from jax.experimental import pallas as pl
from jax.experimental.pallas import tpu as pltpu
from jax.experimental.pallas import tpu_sc as plsc

D_MODEL = 1024
D_FF = 2816
N_EXPERTS = 8
T_TOKENS = 4096
N_ASSIGN = 2 * T_TOKENS

BMM = 512
MAXB = N_ASSIGN // BMM + N_EXPERTS
PADT = MAXB * BMM
BF = 256
NF = D_FF // BF

RCHUNK = 512
NRC = T_TOKENS // RCHUNK

NWORK = 32
TPW = T_TOKENS // NWORK
DCH = 32
CT = 16


def _sc_compiler_params():
    cp = pltpu.CompilerParams()
    if "needs_layout_passes" in pltpu.CompilerParams.__dataclass_fields__:
        cp = dataclasses.replace(cp, needs_layout_passes=False)
    return cp


def _router_body(x_ref, gw_ref, rk_ref, eid_ref, wv_ref, po_ref,
                 be_ref, mb_ref, run_ref, tri_ref):
    c = pl.program_id(0)
    n2 = 2 * RCHUNK

    @pl.when(c == 0)
    def _init():
        run_ref[...] = jnp.zeros((N_EXPERTS, 1), jnp.float32)
        ri = lax.broadcasted_iota(jnp.int32, (n2, n2), 0)
        ci = lax.broadcasted_iota(jnp.int32, (n2, n2), 1)
        tri_ref[...] = (ri < ci).astype(jnp.float32)

    @pl.when(c < NRC)
    def _chunk():
        xb = x_ref[...]
        logits = lax.dot_general(gw_ref[...], xb, (((1,), (1,)), ((), ())),
                                 preferred_element_type=jnp.float32)
        row = lax.broadcasted_iota(jnp.int32, (N_EXPERTS, RCHUNK), 0)
        m1 = jnp.max(logits, axis=0, keepdims=True)
        i1 = jnp.min(jnp.where(logits == m1, row, N_EXPERTS), axis=0,
                     keepdims=True)
        masked = jnp.where(row == i1, -jnp.inf, logits)
        m2 = jnp.max(masked, axis=0, keepdims=True)
        i2 = jnp.min(jnp.where(masked == m2, row, N_EXPERTS), axis=0,
                     keepdims=True)
        wa = 1.0 / (1.0 + jnp.exp(m2 - m1))
        wb = 1.0 - wa

        eid_ref[...] = jnp.concatenate([i1, i2], axis=0)
        wv_ref[...] = jnp.concatenate([wa, wb], axis=0)

        oh_a = (row == i1).astype(jnp.float32)
        oh_b = (row == i2).astype(jnp.float32)
        oh = jnp.concatenate([oh_a, oh_b], axis=1)
        excl = lax.dot_general(oh, tri_ref[...], (((1,), (0,)), ((), ())),
                               preferred_element_type=jnp.float32)
        rank = jnp.sum((excl + run_ref[...]) * oh, axis=0,
                       keepdims=True)
        rk_ref[0:1, :] = rank[:, 0:RCHUNK].astype(jnp.int32)
        rk_ref[1:2, :] = rank[:, RCHUNK:n2].astype(jnp.int32)
        run_ref[...] += jnp.sum(oh, axis=1, keepdims=True)

    @pl.when(c == NRC)
    def _finalize():
        counts = run_ref[...]
        pc = jnp.ceil(counts / BMM) * BMM
        li = lax.broadcasted_iota(jnp.int32, (N_EXPERTS, N_EXPERTS), 0)
        lj = lax.broadcasted_iota(jnp.int32, (N_EXPERTS, N_EXPERTS), 1)
        lower = (lj < li).astype(jnp.float32)
        pad_off = lax.dot_general(lower, pc, (((1,), (0,)), ((), ())),
                                  preferred_element_type=jnp.float32)
        po_ref[...] = jnp.concatenate(
            [pad_off, jnp.zeros((N_EXPERTS, 1), jnp.float32)],
            axis=0).astype(jnp.int32)

        total = jnp.sum(pc, axis=0, keepdims=True)
        nact = (total / BMM).astype(jnp.int32)
        b_iota = lax.broadcasted_iota(jnp.int32, (1, MAXB), 1)
        b_eff = jnp.minimum(b_iota, nact - 1)
        po_l = jnp.broadcast_to(pad_off, (N_EXPERTS, MAXB))
        cmp = (b_eff.astype(jnp.float32) * BMM >= po_l).astype(jnp.float32)
        be = jnp.sum(cmp, axis=0, keepdims=True).astype(jnp.int32) - 1
        be_ref[...] = be
        mb_ref[...] = b_eff


@functools.partial(jax.jit, static_argnames=("interpret",))
def _router(xr, gate_w, interpret=False):
    return pl.pallas_call(
        _router_body,
        grid=(NRC + 1,),
        in_specs=[
            pl.BlockSpec((RCHUNK, D_MODEL),
                         lambda c: (jnp.minimum(c, NRC - 1), 0)),
            pl.BlockSpec((N_EXPERTS, D_MODEL), lambda c: (0, 0)),
        ],
        out_specs=[
            pl.BlockSpec((2, RCHUNK), lambda c: (0, jnp.minimum(c, NRC - 1))),
            pl.BlockSpec((2, RCHUNK), lambda c: (0, jnp.minimum(c, NRC - 1))),
            pl.BlockSpec((2, RCHUNK), lambda c: (0, jnp.minimum(c, NRC - 1))),
            pl.BlockSpec((2 * N_EXPERTS, 1), lambda c: (0, 0)),
            pl.BlockSpec((1, MAXB), lambda c: (0, 0)),
            pl.BlockSpec((1, MAXB), lambda c: (0, 0)),
        ],
        out_shape=[
            jax.ShapeDtypeStruct((2, T_TOKENS), jnp.int32),
            jax.ShapeDtypeStruct((2, T_TOKENS), jnp.int32),
            jax.ShapeDtypeStruct((2, T_TOKENS), jnp.float32),
            jax.ShapeDtypeStruct((2 * N_EXPERTS, 1), jnp.int32),
            jax.ShapeDtypeStruct((1, MAXB), jnp.int32),
            jax.ShapeDtypeStruct((1, MAXB), jnp.int32),
        ],
        scratch_shapes=[
            pltpu.VMEM((N_EXPERTS, 1), jnp.float32),
            pltpu.VMEM((2 * RCHUNK, 2 * RCHUNK), jnp.float32),
        ],
        compiler_params=pltpu.CompilerParams(
            dimension_semantics=("arbitrary",)),
        interpret=interpret,
    )(xr, gate_w)


def _sc_dispatch(rk2, eid2, po16, xr):
    mesh = plsc.VectorSubcoreMesh(core_axis_name="c", subcore_axis_name="s")
    nds = TPW // DCH

    @functools.partial(
        pl.kernel, mesh=mesh,
        out_type=jax.ShapeDtypeStruct((PADT, D_MODEL), jnp.float32),
        scratch_types=[
            pltpu.VMEM((nds, DCH), jnp.int32),
            pltpu.VMEM((nds, DCH), jnp.int32),
            pltpu.VMEM((nds, DCH), jnp.int32),
            pltpu.VMEM((nds, DCH), jnp.int32),
            pltpu.VMEM((16,), jnp.int32),
            pltpu.VMEM((2, DCH, D_MODEL), jnp.float32),
            pltpu.SemaphoreType.DMA,
            pltpu.SemaphoreType.DMA,
            pltpu.SemaphoreType.DMA,
            pltpu.SemaphoreType.DMA,
            pltpu.SemaphoreType.DMA,
            pltpu.SemaphoreType.DMA,
        ],
        compiler_params=_sc_compiler_params(),
    )
    def k(rk_hbm, eid_hbm, po_hbm, x_hbm, xs_hbm,
          idxa_v, idxb_v, eida_v, eidb_v, po_v, buf,
          sl0, sl1, sa0, sa1, sb0, sb1):
        cid = lax.axis_index("c")
        sid = lax.axis_index("s")
        w = cid * 16 + sid
        t0 = w * TPW
        pltpu.sync_copy(po_hbm, po_v)
        for j in range(nds):
            b0 = t0 + j * DCH
            pltpu.sync_copy(rk_hbm.at[0, pl.ds(b0, DCH)], idxa_v.at[j])
            pltpu.sync_copy(rk_hbm.at[1, pl.ds(b0, DCH)], idxb_v.at[j])
            pltpu.sync_copy(eid_hbm.at[0, pl.ds(b0, DCH)], eida_v.at[j])
            pltpu.sync_copy(eid_hbm.at[1, pl.ds(b0, DCH)], eidb_v.at[j])

        for j in range(nds):
            for i in range(0, DCH, 16):
                sl = pl.ds(i, 16)
                idxa_v[j, sl] += plsc.load_gather(po_v, [eida_v[j, sl]])
                idxb_v[j, sl] += plsc.load_gather(po_v, [eidb_v[j, sl]])

        sls = (sl0, sl1)
        sas = (sa0, sa1)
        sbs = (sb0, sb1)
        pend = {}

        def issue_load(j):
            b = j & 1
            pend[("l", b)] = pltpu.async_copy(
                x_hbm.at[pl.ds(t0 + j * DCH, DCH)], buf.at[b], sls[b])

        issue_load(0)
        for j in range(nds):
            b = j & 1
            pend[("l", b)].wait()
            if j + 1 < nds:
                if ("a", 1 - b) in pend:
                    pend.pop(("a", 1 - b)).wait()
                    pend.pop(("b", 1 - b)).wait()
                issue_load(j + 1)
            pend[("a", b)] = pltpu.async_copy(
                buf.at[b], xs_hbm.at[idxa_v.at[j]], sas[b])
            pend[("b", b)] = pltpu.async_copy(
                buf.at[b], xs_hbm.at[idxb_v.at[j]], sbs[b])
        for key in list(pend):
            if key[0] in ("a", "b"):
                pend[key].wait()

    return k(rk2, eid2, po16, xr)


def _gmm_body(be_ref, mb_ref, xs_ref, w1_ref, w3_ref, w2_ref, out_ref,
              xb_ref, h_ref, w2b_ref):
    m = pl.program_id(0)
    f = pl.program_id(1)
    active = m == mb_ref[0, m]

    @pl.when(active)
    def _compute():
        @pl.when(f == 0)
        def _cast():
            xb_ref[...] = xs_ref[...].astype(jnp.bfloat16)

        xb = xb_ref[...]
        w1b = w1_ref[0].astype(jnp.bfloat16)
        w3b = w3_ref[0].astype(jnp.bfloat16)
        h1 = lax.dot_general(xb, w1b, (((1,), (1,)), ((), ())),
                             preferred_element_type=jnp.float32)
        h3 = lax.dot_general(xb, w3b, (((1,), (1,)), ((), ())),
                             preferred_element_type=jnp.float32)
        h = ((h1 * lax.logistic(h1)) * h3).astype(jnp.bfloat16)
        fsl = pl.ds(pl.multiple_of(f * BF, BF), BF)
        h_ref[:, fsl] = h
        w2b_ref[:, fsl] = w2_ref[0, :, fsl].astype(jnp.bfloat16)

        @pl.when(f == NF - 1)
        def _big_dot():
            out_ref[...] = lax.dot_general(
                h_ref[...], w2b_ref[...], (((1,), (1,)), ((), ())),
                preferred_element_type=jnp.float32)


@functools.partial(jax.jit, static_argnames=("interpret",))
def _gmm(be, mb, xs, w1, w3, w2, interpret=False):
    def f_eff(m, f, be_r, mb_r):
        return jnp.where(m == mb_r[0, m], f, 0)

    grid_spec = pltpu.PrefetchScalarGridSpec(
        num_scalar_prefetch=2,
        grid=(MAXB, NF),
        in_specs=[
            pl.BlockSpec((BMM, D_MODEL),
                         lambda m, f, be_r, mb_r: (mb_r[0, m], 0)),
            pl.BlockSpec((1, BF, D_MODEL),
                         lambda m, f, be_r, mb_r: (be_r[0, m],
                                                   f_eff(m, f, be_r, mb_r),
                                                   0)),
            pl.BlockSpec((1, BF, D_MODEL),
                         lambda m, f, be_r, mb_r: (be_r[0, m],
                                                   f_eff(m, f, be_r, mb_r),
                                                   0)),
            pl.BlockSpec((1, D_MODEL, D_FF),
                         lambda m, f, be_r, mb_r: (be_r[0, m], 0, 0)),
        ],
        out_specs=pl.BlockSpec((BMM, D_MODEL),
                               lambda m, f, be_r, mb_r: (mb_r[0, m], 0)),
        scratch_shapes=[
            pltpu.VMEM((BMM, D_MODEL), jnp.bfloat16),
            pltpu.VMEM((BMM, D_FF), jnp.bfloat16),
            pltpu.VMEM((D_MODEL, D_FF), jnp.bfloat16),
        ],
    )
    return pl.pallas_call(
        _gmm_body,
        grid_spec=grid_spec,
        out_shape=jax.ShapeDtypeStruct((PADT, D_MODEL), jnp.float32),
        compiler_params=pltpu.CompilerParams(
            dimension_semantics=("arbitrary", "arbitrary")),
        interpret=interpret,
    )(be, mb, xs, w1, w3, w2)


def _sc_combine(y, rk2, eid2, po16, wv2):
    mesh = plsc.VectorSubcoreMesh(core_axis_name="c", subcore_axis_name="s")

    @functools.partial(
        pl.kernel, mesh=mesh,
        out_type=jax.ShapeDtypeStruct((T_TOKENS, D_MODEL), jnp.float32),
        scratch_types=[
            pltpu.VMEM((TPW,), jnp.int32),
            pltpu.VMEM((TPW,), jnp.int32),
            pltpu.VMEM((TPW,), jnp.int32),
            pltpu.VMEM((TPW,), jnp.int32),
            pltpu.VMEM((16,), jnp.int32),
            pltpu.VMEM((TPW,), jnp.float32),
            pltpu.VMEM((TPW,), jnp.float32),
            pltpu.VMEM((2, CT, D_MODEL), jnp.float32),
            pltpu.VMEM((2, CT, D_MODEL), jnp.float32),
            pltpu.VMEM((2, CT, D_MODEL), jnp.float32),
            pltpu.SemaphoreType.DMA,
            pltpu.SemaphoreType.DMA,
            pltpu.SemaphoreType.DMA,
            pltpu.SemaphoreType.DMA,
            pltpu.SemaphoreType.DMA,
            pltpu.SemaphoreType.DMA,
        ],
        compiler_params=_sc_compiler_params(),
    )
    def k(y_hbm, rk_hbm, eid_hbm, po_hbm, wv_hbm, out_hbm,
          idxa_v, idxb_v, eida_v, eidb_v, po_v, wva_v, wvb_v,
          bufa, bufb, obuf, sa0, sa1, sb0, sb1, so0, so1):
        cid = lax.axis_index("c")
        sid = lax.axis_index("s")
        w = cid * 16 + sid
        t0 = w * TPW
        pltpu.sync_copy(po_hbm, po_v)
        pltpu.sync_copy(rk_hbm.at[0, pl.ds(t0, TPW)], idxa_v)
        pltpu.sync_copy(rk_hbm.at[1, pl.ds(t0, TPW)], idxb_v)
        pltpu.sync_copy(eid_hbm.at[0, pl.ds(t0, TPW)], eida_v)
        pltpu.sync_copy(eid_hbm.at[1, pl.ds(t0, TPW)], eidb_v)
        pltpu.sync_copy(wv_hbm.at[0, pl.ds(t0, TPW)], wva_v)
        pltpu.sync_copy(wv_hbm.at[1, pl.ds(t0, TPW)], wvb_v)

        for i in range(0, TPW, 16):
            sl = pl.ds(i, 16)
            idxa_v[sl] += plsc.load_gather(po_v, [eida_v[sl]])
            idxb_v[sl] += plsc.load_gather(po_v, [eidb_v[sl]])

        nsub = TPW // CT
        sas = (sa0, sa1)
        sbs = (sb0, sb1)
        sos = (so0, so1)
        pend = {}

        def issue_gathers(sub):
            b = sub & 1
            pend[("a", b)] = pltpu.async_copy(
                y_hbm.at[idxa_v.at[pl.ds(sub * CT, CT)]], bufa.at[b], sas[b])
            pend[("b", b)] = pltpu.async_copy(
                y_hbm.at[idxb_v.at[pl.ds(sub * CT, CT)]], bufb.at[b], sbs[b])

        issue_gathers(0)
        for sub in range(nsub):
            b = sub & 1
            pend[("a", b)].wait()
            pend[("b", b)].wait()
            if sub + 1 < nsub:
                issue_gathers(sub + 1)
            if sub >= 2:
                pend[("o", b)].wait()

            @pl.loop(0, CT)
            def _row(i):
                wa = plsc.load_gather(
                    wva_v, [jnp.full((16,), sub * CT + i, jnp.int32)])
                wb = plsc.load_gather(
                    wvb_v, [jnp.full((16,), sub * CT + i, jnp.int32)])

                @pl.loop(0, D_MODEL, step=64)
                def _lane(j):
                    for u in range(4):
                        sl = pl.ds(j + u * 16, 16)
                        obuf[b, i, sl] = (wa * bufa[b, i, sl]
                                          + wb * bufb[b, i, sl])

            pend[("o", b)] = pltpu.async_copy(
                obuf.at[b], out_hbm.at[pl.ds(t0 + sub * CT, CT)], sos[b])
        pend[("o", (nsub - 1) & 1)].wait()
        pend[("o", nsub & 1)].wait()

    return k(y, rk2, eid2, po16, wv2)


def kernel(x, gate_w, w1, w3, w2):
    B, S, D = x.shape
    xr = x.reshape(-1, D)
    rk2, eid2, wv2, po, be, mb = _router(xr, gate_w)
    po16 = po.reshape(-1)
    xs = _sc_dispatch(rk2, eid2, po16, xr)
    y = _gmm(be, mb, xs, w1, w3, w2)
    out = _sc_combine(y, rk2, eid2, po16, wv2)
    return out.reshape(B, S, D)


# speedup vs baseline: 1.0055x; 1.0055x over previous
import dataclasses
import functools

import jax
import jax.numpy as jnp
from jax import lax
from jax.experimental import pallas as pl
from jax.experimental.pallas import tpu as pltpu
from jax.experimental.pallas import tpu_sc as plsc

D_MODEL = 1024
D_FF = 2816
N_EXPERTS = 8
T_TOKENS = 4096
N_ASSIGN = 2 * T_TOKENS

BMM = 512
MAXB = N_ASSIGN // BMM + N_EXPERTS
PADT = MAXB * BMM
BF = 256
NF = D_FF // BF

RCHUNK = 512
NRC = T_TOKENS // RCHUNK

NWORK = 32
TPW = T_TOKENS // NWORK
DCH = 64
CT = 16


def _sc_compiler_params():
    cp = pltpu.CompilerParams()
    if "needs_layout_passes" in pltpu.CompilerParams.__dataclass_fields__:
        cp = dataclasses.replace(cp, needs_layout_passes=False)
    return cp


def _router_body(x_ref, gw_ref, rk_ref, eid_ref, wv_ref, po_ref,
                 be_ref, mb_ref, run_ref, tri_ref):
    c = pl.program_id(0)
    n2 = 2 * RCHUNK

    @pl.when(c == 0)
    def _init():
        run_ref[...] = jnp.zeros((N_EXPERTS, 1), jnp.float32)
        ri = lax.broadcasted_iota(jnp.int32, (n2, n2), 0)
        ci = lax.broadcasted_iota(jnp.int32, (n2, n2), 1)
        tri_ref[...] = (ri < ci).astype(jnp.float32)

    @pl.when(c < NRC)
    def _chunk():
        xb = x_ref[...]
        logits = lax.dot_general(gw_ref[...], xb, (((1,), (1,)), ((), ())),
                                 preferred_element_type=jnp.float32)
        row = lax.broadcasted_iota(jnp.int32, (N_EXPERTS, RCHUNK), 0)
        m1 = jnp.max(logits, axis=0, keepdims=True)
        i1 = jnp.min(jnp.where(logits == m1, row, N_EXPERTS), axis=0,
                     keepdims=True)
        masked = jnp.where(row == i1, -jnp.inf, logits)
        m2 = jnp.max(masked, axis=0, keepdims=True)
        i2 = jnp.min(jnp.where(masked == m2, row, N_EXPERTS), axis=0,
                     keepdims=True)
        wa = 1.0 / (1.0 + jnp.exp(m2 - m1))
        wb = 1.0 - wa

        eid_ref[...] = jnp.concatenate([i1, i2], axis=0)
        wv_ref[...] = jnp.concatenate([wa, wb], axis=0)

        oh_a = (row == i1).astype(jnp.float32)
        oh_b = (row == i2).astype(jnp.float32)
        oh = jnp.concatenate([oh_a, oh_b], axis=1)
        excl = lax.dot_general(oh, tri_ref[...], (((1,), (0,)), ((), ())),
                               preferred_element_type=jnp.float32)
        rank = jnp.sum((excl + run_ref[...]) * oh, axis=0,
                       keepdims=True)
        rk_ref[0:1, :] = rank[:, 0:RCHUNK].astype(jnp.int32)
        rk_ref[1:2, :] = rank[:, RCHUNK:n2].astype(jnp.int32)
        run_ref[...] += jnp.sum(oh, axis=1, keepdims=True)

    @pl.when(c == NRC)
    def _finalize():
        counts = run_ref[...]
        pc = jnp.ceil(counts / BMM) * BMM
        li = lax.broadcasted_iota(jnp.int32, (N_EXPERTS, N_EXPERTS), 0)
        lj = lax.broadcasted_iota(jnp.int32, (N_EXPERTS, N_EXPERTS), 1)
        lower = (lj < li).astype(jnp.float32)
        pad_off = lax.dot_general(lower, pc, (((1,), (0,)), ((), ())),
                                  preferred_element_type=jnp.float32)
        po_ref[...] = jnp.concatenate(
            [pad_off, jnp.zeros((N_EXPERTS, 1), jnp.float32)],
            axis=0).astype(jnp.int32)

        total = jnp.sum(pc, axis=0, keepdims=True)
        nact = (total / BMM).astype(jnp.int32)
        b_iota = lax.broadcasted_iota(jnp.int32, (1, MAXB), 1)
        b_eff = jnp.minimum(b_iota, nact - 1)
        po_l = jnp.broadcast_to(pad_off, (N_EXPERTS, MAXB))
        cmp = (b_eff.astype(jnp.float32) * BMM >= po_l).astype(jnp.float32)
        be = jnp.sum(cmp, axis=0, keepdims=True).astype(jnp.int32) - 1
        be_ref[...] = be
        mb_ref[...] = b_eff


@functools.partial(jax.jit, static_argnames=("interpret",))
def _router(xr, gate_w, interpret=False):
    return pl.pallas_call(
        _router_body,
        grid=(NRC + 1,),
        in_specs=[
            pl.BlockSpec((RCHUNK, D_MODEL),
                         lambda c: (jnp.minimum(c, NRC - 1), 0)),
            pl.BlockSpec((N_EXPERTS, D_MODEL), lambda c: (0, 0)),
        ],
        out_specs=[
            pl.BlockSpec((2, RCHUNK), lambda c: (0, jnp.minimum(c, NRC - 1))),
            pl.BlockSpec((2, RCHUNK), lambda c: (0, jnp.minimum(c, NRC - 1))),
            pl.BlockSpec((2, RCHUNK), lambda c: (0, jnp.minimum(c, NRC - 1))),
            pl.BlockSpec((2 * N_EXPERTS, 1), lambda c: (0, 0)),
            pl.BlockSpec((1, MAXB), lambda c: (0, 0)),
            pl.BlockSpec((1, MAXB), lambda c: (0, 0)),
        ],
        out_shape=[
            jax.ShapeDtypeStruct((2, T_TOKENS), jnp.int32),
            jax.ShapeDtypeStruct((2, T_TOKENS), jnp.int32),
            jax.ShapeDtypeStruct((2, T_TOKENS), jnp.float32),
            jax.ShapeDtypeStruct((2 * N_EXPERTS, 1), jnp.int32),
            jax.ShapeDtypeStruct((1, MAXB), jnp.int32),
            jax.ShapeDtypeStruct((1, MAXB), jnp.int32),
        ],
        scratch_shapes=[
            pltpu.VMEM((N_EXPERTS, 1), jnp.float32),
            pltpu.VMEM((2 * RCHUNK, 2 * RCHUNK), jnp.float32),
        ],
        compiler_params=pltpu.CompilerParams(
            dimension_semantics=("arbitrary",)),
        interpret=interpret,
    )(xr, gate_w)


def _sc_dispatch(rk2, eid2, po16, xr):
    mesh = plsc.VectorSubcoreMesh(core_axis_name="c", subcore_axis_name="s")
    nds = TPW // DCH

    @functools.partial(
        pl.kernel, mesh=mesh,
        out_type=jax.ShapeDtypeStruct((PADT, D_MODEL), jnp.float32),
        scratch_types=[
            pltpu.VMEM((nds, DCH), jnp.int32),
            pltpu.VMEM((nds, DCH), jnp.int32),
            pltpu.VMEM((nds, DCH), jnp.int32),
            pltpu.VMEM((nds, DCH), jnp.int32),
            pltpu.VMEM((16,), jnp.int32),
            pltpu.VMEM((DCH, D_MODEL), jnp.float32),
            pltpu.SemaphoreType.DMA,
            pltpu.SemaphoreType.DMA,
        ],
        compiler_params=_sc_compiler_params(),
    )
    def k(rk_hbm, eid_hbm, po_hbm, x_hbm, xs_hbm,
          idxa_v, idxb_v, eida_v, eidb_v, po_v, buf, sema, semb):
        cid = lax.axis_index("c")
        sid = lax.axis_index("s")
        w = cid * 16 + sid
        t0 = w * TPW
        pltpu.sync_copy(po_hbm, po_v)
        for j in range(nds):
            b0 = t0 + j * DCH
            pltpu.sync_copy(rk_hbm.at[0, pl.ds(b0, DCH)], idxa_v.at[j])
            pltpu.sync_copy(rk_hbm.at[1, pl.ds(b0, DCH)], idxb_v.at[j])
            pltpu.sync_copy(eid_hbm.at[0, pl.ds(b0, DCH)], eida_v.at[j])
            pltpu.sync_copy(eid_hbm.at[1, pl.ds(b0, DCH)], eidb_v.at[j])

        for j in range(nds):
            for i in range(0, DCH, 16):
                sl = pl.ds(i, 16)
                idxa_v[j, sl] += plsc.load_gather(po_v, [eida_v[j, sl]])
                idxb_v[j, sl] += plsc.load_gather(po_v, [eidb_v[j, sl]])

        for j in range(nds):
            pltpu.sync_copy(x_hbm.at[pl.ds(t0 + j * DCH, DCH)], buf)
            ca = pltpu.async_copy(buf, xs_hbm.at[idxa_v.at[j]], sema)
            cb = pltpu.async_copy(buf, xs_hbm.at[idxb_v.at[j]], semb)
            ca.wait()
            cb.wait()

    return k(rk2, eid2, po16, xr)


def _gmm_body(be_ref, mb_ref, xs_ref, w1_ref, w3_ref, w2_ref, out_ref,
              xb_ref, h_ref, w2b_ref):
    m = pl.program_id(0)
    f = pl.program_id(1)
    active = m == mb_ref[0, m]

    @pl.when(active)
    def _compute():
        @pl.when(f == 0)
        def _cast():
            xb_ref[...] = xs_ref[...].astype(jnp.bfloat16)

        xb = xb_ref[...]
        w1b = w1_ref[0].astype(jnp.bfloat16)
        w3b = w3_ref[0].astype(jnp.bfloat16)
        h1 = lax.dot_general(xb, w1b, (((1,), (1,)), ((), ())),
                             preferred_element_type=jnp.float32)
        h3 = lax.dot_general(xb, w3b, (((1,), (1,)), ((), ())),
                             preferred_element_type=jnp.float32)
        h = ((h1 * lax.logistic(h1)) * h3).astype(jnp.bfloat16)
        fsl = pl.ds(pl.multiple_of(f * BF, BF), BF)
        h_ref[:, fsl] = h
        w2b_ref[:, fsl] = w2_ref[0, :, fsl].astype(jnp.bfloat16)

        @pl.when(f == NF - 1)
        def _big_dot():
            out_ref[...] = lax.dot_general(
                h_ref[...], w2b_ref[...], (((1,), (1,)), ((), ())),
                preferred_element_type=jnp.float32)


@functools.partial(jax.jit, static_argnames=("interpret",))
def _gmm(be, mb, xs, w1, w3, w2, interpret=False):
    def f_eff(m, f, be_r, mb_r):
        return jnp.where(m == mb_r[0, m], f, 0)

    grid_spec = pltpu.PrefetchScalarGridSpec(
        num_scalar_prefetch=2,
        grid=(MAXB, NF),
        in_specs=[
            pl.BlockSpec((BMM, D_MODEL),
                         lambda m, f, be_r, mb_r: (mb_r[0, m], 0)),
            pl.BlockSpec((1, BF, D_MODEL),
                         lambda m, f, be_r, mb_r: (be_r[0, m],
                                                   f_eff(m, f, be_r, mb_r),
                                                   0)),
            pl.BlockSpec((1, BF, D_MODEL),
                         lambda m, f, be_r, mb_r: (be_r[0, m],
                                                   f_eff(m, f, be_r, mb_r),
                                                   0)),
            pl.BlockSpec((1, D_MODEL, D_FF),
                         lambda m, f, be_r, mb_r: (be_r[0, m], 0, 0)),
        ],
        out_specs=pl.BlockSpec((BMM, D_MODEL),
                               lambda m, f, be_r, mb_r: (mb_r[0, m], 0)),
        scratch_shapes=[
            pltpu.VMEM((BMM, D_MODEL), jnp.bfloat16),
            pltpu.VMEM((BMM, D_FF), jnp.bfloat16),
            pltpu.VMEM((D_MODEL, D_FF), jnp.bfloat16),
        ],
    )
    return pl.pallas_call(
        _gmm_body,
        grid_spec=grid_spec,
        out_shape=jax.ShapeDtypeStruct((PADT, D_MODEL), jnp.float32),
        compiler_params=pltpu.CompilerParams(
            dimension_semantics=("arbitrary", "arbitrary")),
        interpret=interpret,
    )(be, mb, xs, w1, w3, w2)


def _sc_combine(y, rk2, eid2, po16, wv2):
    mesh = plsc.VectorSubcoreMesh(core_axis_name="c", subcore_axis_name="s")

    @functools.partial(
        pl.kernel, mesh=mesh,
        out_type=jax.ShapeDtypeStruct((T_TOKENS, D_MODEL), jnp.float32),
        scratch_types=[
            pltpu.VMEM((TPW,), jnp.int32),
            pltpu.VMEM((TPW,), jnp.int32),
            pltpu.VMEM((TPW,), jnp.int32),
            pltpu.VMEM((TPW,), jnp.int32),
            pltpu.VMEM((16,), jnp.int32),
            pltpu.VMEM((TPW,), jnp.float32),
            pltpu.VMEM((TPW,), jnp.float32),
            pltpu.VMEM((2, CT, D_MODEL), jnp.float32),
            pltpu.VMEM((2, CT, D_MODEL), jnp.float32),
            pltpu.VMEM((2, CT, D_MODEL), jnp.float32),
            pltpu.SemaphoreType.DMA,
            pltpu.SemaphoreType.DMA,
            pltpu.SemaphoreType.DMA,
            pltpu.SemaphoreType.DMA,
            pltpu.SemaphoreType.DMA,
            pltpu.SemaphoreType.DMA,
        ],
        compiler_params=_sc_compiler_params(),
    )
    def k(y_hbm, rk_hbm, eid_hbm, po_hbm, wv_hbm, out_hbm,
          idxa_v, idxb_v, eida_v, eidb_v, po_v, wva_v, wvb_v,
          bufa, bufb, obuf, sa0, sa1, sb0, sb1, so0, so1):
        cid = lax.axis_index("c")
        sid = lax.axis_index("s")
        w = cid * 16 + sid
        t0 = w * TPW
        pltpu.sync_copy(po_hbm, po_v)
        pltpu.sync_copy(rk_hbm.at[0, pl.ds(t0, TPW)], idxa_v)
        pltpu.sync_copy(rk_hbm.at[1, pl.ds(t0, TPW)], idxb_v)
        pltpu.sync_copy(eid_hbm.at[0, pl.ds(t0, TPW)], eida_v)
        pltpu.sync_copy(eid_hbm.at[1, pl.ds(t0, TPW)], eidb_v)
        pltpu.sync_copy(wv_hbm.at[0, pl.ds(t0, TPW)], wva_v)
        pltpu.sync_copy(wv_hbm.at[1, pl.ds(t0, TPW)], wvb_v)

        for i in range(0, TPW, 16):
            sl = pl.ds(i, 16)
            idxa_v[sl] += plsc.load_gather(po_v, [eida_v[sl]])
            idxb_v[sl] += plsc.load_gather(po_v, [eidb_v[sl]])

        nsub = TPW // CT
        sas = (sa0, sa1)
        sbs = (sb0, sb1)
        sos = (so0, so1)
        pend = {}

        def issue_gathers(sub):
            b = sub & 1
            pend[("a", b)] = pltpu.async_copy(
                y_hbm.at[idxa_v.at[pl.ds(sub * CT, CT)]], bufa.at[b], sas[b])
            pend[("b", b)] = pltpu.async_copy(
                y_hbm.at[idxb_v.at[pl.ds(sub * CT, CT)]], bufb.at[b], sbs[b])

        issue_gathers(0)
        for sub in range(nsub):
            b = sub & 1
            pend[("a", b)].wait()
            pend[("b", b)].wait()
            if sub + 1 < nsub:
                issue_gathers(sub + 1)
            if sub >= 2:
                pend[("o", b)].wait()

            @pl.loop(0, CT)
            def _row(i):
                wa = plsc.load_gather(
                    wva_v, [jnp.full((16,), sub * CT + i, jnp.int32)])
                wb = plsc.load_gather(
                    wvb_v, [jnp.full((16,), sub * CT + i, jnp.int32)])

                @pl.loop(0, D_MODEL, step=64)
                def _lane(j):
                    for u in range(4):
                        sl = pl.ds(j + u * 16, 16)
                        obuf[b, i, sl] = (wa * bufa[b, i, sl]
                                          + wb * bufb[b, i, sl])

            pend[("o", b)] = pltpu.async_copy(
                obuf.at[b], out_hbm.at[pl.ds(t0 + sub * CT, CT)], sos[b])
        pend[("o", (nsub - 1) & 1)].wait()
        pend[("o", nsub & 1)].wait()

    return k(y, rk2, eid2, po16, wv2)


def kernel(x, gate_w, w1, w3, w2):
    B, S, D = x.shape
    xr = x.reshape(-1, D)
    rk2, eid2, wv2, po, be, mb = _router(xr, gate_w)
    po16 = po.reshape(-1)
    xs = _sc_dispatch(rk2, eid2, po16, xr)
    y = _gmm(be, mb, xs, w1, w3, w2)
    out = _sc_combine(y, rk2, eid2, po16, wv2)
    return out.reshape(B, S, D)
